# baseline (speedup 1.0000x reference)
import functools

import jax
import jax.numpy as jnp
from jax import lax
from jax.experimental import pallas as pl
from jax.experimental.pallas import tpu as pltpu
from jax.experimental.pallas import tpu_sc as plsc

_SYN = 4
_L = 16
_NW = 32
_CHUNK = 128
_T = 200
_NBKG = 100


def sc_spmm(rob_flat, cols_flat, w_flat, n_out, npad):
    slab = npad // _NW
    syn_slab = _SYN * slab
    chunks = slab // _CHUNK
    mesh = plsc.VectorSubcoreMesh(core_axis_name="c", subcore_axis_name="s")

    @functools.partial(
        pl.kernel,
        mesh=mesh,
        out_type=jax.ShapeDtypeStruct((_T, n_out), jnp.float32),
        scratch_types=[
            pltpu.VMEM((_T * _NBKG,), jnp.float32),
            pltpu.VMEM((syn_slab,), jnp.int32),
            pltpu.VMEM((syn_slab,), jnp.float32),
            pltpu.VMEM((_T, _CHUNK), jnp.float32),
        ],
        compiler_params=pltpu.CompilerParams(needs_layout_passes=False,
                                             use_tc_tiling_on_sc=False),
    )
    def k(rob_hbm, cols_hbm, w_hbm, out_hbm, rob_v, cols_v, w_v, out_v):
        wid = lax.axis_index("s") * 2 + lax.axis_index("c")
        base_syn = wid * syn_slab
        pltpu.sync_copy(rob_hbm, rob_v)
        pltpu.sync_copy(cols_hbm.at[pl.ds(base_syn, syn_slab)], cols_v)
        pltpu.sync_copy(w_hbm.at[pl.ds(base_syn, syn_slab)], w_v)
        lanes = lax.iota(jnp.int32, _L)
        base_n = wid * slab

        def compute_chunk(moff):
            for i in range(_CHUNK // _L):
                idx4 = (moff + i * _L + lanes) * _SYN
                cs = [plsc.load_gather(cols_v, [idx4 + s])
                      for s in range(_SYN)]
                ws = [plsc.load_gather(w_v, [idx4 + s])
                      for s in range(_SYN)]

                def t_body(t, _):
                    tb = t * _NBKG
                    acc = plsc.load_gather(rob_v, [cs[0] + tb]) * ws[0]
                    acc += plsc.load_gather(rob_v, [cs[1] + tb]) * ws[1]
                    acc += plsc.load_gather(rob_v, [cs[2] + tb]) * ws[2]
                    acc += plsc.load_gather(rob_v, [cs[3] + tb]) * ws[3]
                    out_v[t, pl.ds(i * _L, _L)] = acc
                    return 0

                lax.fori_loop(0, _T, t_body, 0)

        tail = n_out % _CHUNK

        def do_chunk(c, _):
            base = pl.multiple_of(base_n + c * _CHUNK, _CHUNK)

            @pl.when(base + _CHUNK <= n_out)
            def _():
                compute_chunk(base - base_n)
                pltpu.sync_copy(out_v, out_hbm.at[:, pl.ds(base, _CHUNK)])

            if tail:
                @pl.when(base == n_out - tail)
                def _():
                    compute_chunk(base - base_n)
                    pltpu.sync_copy(
                        out_v.at[:, pl.ds(0, tail)],
                        out_hbm.at[:, pl.ds(base, tail)])
            return 0

        lax.fori_loop(0, chunks, do_chunk, 0)

    return k(rob_flat, cols_flat, w_flat)


def kernel(inp, rest_of_brain, w_v1, idx_v1, w_lm, idx_lm):
    t, nbkg = rest_of_brain.shape
    cols = jnp.concatenate([idx_v1[:, 1], idx_lm[:, 1]])
    w = jnp.concatenate([w_v1, w_lm])
    n = cols.shape[0] // _SYN
    npad = pl.cdiv(n, _NW * _CHUNK) * _NW * _CHUNK
    cols_flat = jnp.pad(cols, (0, _SYN * (npad - n)))
    w_flat = jnp.pad(w, (0, _SYN * (npad - n)))
    out = sc_spmm(rest_of_brain.reshape(-1), cols_flat, w_flat, n, npad)
    return out.reshape(1, t, n)


# speedup vs baseline: 1.0141x; 1.0141x over previous
import functools

import jax
import jax.numpy as jnp
from jax import lax
from jax.experimental import pallas as pl
from jax.experimental.pallas import tpu as pltpu
from jax.experimental.pallas import tpu_sc as plsc

_SYN = 4
_L = 16
_NW = 32
_CHUNK = 128
_T = 200
_NBKG = 100


def sc_spmm(rob_flat, cols_flat, w_flat, n_out, npad):
    slab = npad // _NW
    syn_slab = _SYN * slab
    chunks = slab // _CHUNK
    mesh = plsc.VectorSubcoreMesh(core_axis_name="c", subcore_axis_name="s")

    @functools.partial(
        pl.kernel,
        mesh=mesh,
        out_type=jax.ShapeDtypeStruct((_T, n_out), jnp.float32),
        scratch_types=[
            pltpu.VMEM((_T * _NBKG,), jnp.float32),
            pltpu.VMEM((syn_slab,), jnp.int32),
            pltpu.VMEM((syn_slab,), jnp.float32),
            pltpu.VMEM((_T, _CHUNK), jnp.float32),
        ],
        compiler_params=pltpu.CompilerParams(needs_layout_passes=False,
                                             use_tc_tiling_on_sc=False),
    )
    def k(rob_hbm, cols_hbm, w_hbm, out_hbm, rob_v, cols_v, w_v, out_v):
        wid = lax.axis_index("s") * 2 + lax.axis_index("c")
        base_syn = wid * syn_slab
        pltpu.sync_copy(rob_hbm, rob_v)
        pltpu.sync_copy(cols_hbm.at[pl.ds(base_syn, syn_slab)], cols_v)
        pltpu.sync_copy(w_hbm.at[pl.ds(base_syn, syn_slab)], w_v)
        lanes = lax.iota(jnp.int32, _L)
        base_n = wid * slab

        def compute_chunk(moff):
            for i in range(_CHUNK // _L):
                idx4 = (moff + i * _L + lanes) * _SYN
                cs = [plsc.load_gather(cols_v, [idx4 + s])
                      for s in range(_SYN)]
                ws = [plsc.load_gather(w_v, [idx4 + s])
                      for s in range(_SYN)]

                def t_body(tg, _):
                    t0 = tg * 8
                    for u in range(8):
                        tb = (t0 + u) * _NBKG
                        acc = plsc.load_gather(rob_v, [cs[0] + tb]) * ws[0]
                        acc += plsc.load_gather(rob_v, [cs[1] + tb]) * ws[1]
                        acc += plsc.load_gather(rob_v, [cs[2] + tb]) * ws[2]
                        acc += plsc.load_gather(rob_v, [cs[3] + tb]) * ws[3]
                        out_v[t0 + u, pl.ds(i * _L, _L)] = acc
                    return 0

                lax.fori_loop(0, _T // 8, t_body, 0)

        tail = n_out % _CHUNK

        def do_chunk(c, _):
            base = pl.multiple_of(base_n + c * _CHUNK, _CHUNK)

            @pl.when(base + _CHUNK <= n_out)
            def _():
                compute_chunk(base - base_n)
                pltpu.sync_copy(out_v, out_hbm.at[:, pl.ds(base, _CHUNK)])

            if tail:
                @pl.when(base == n_out - tail)
                def _():
                    compute_chunk(base - base_n)
                    pltpu.sync_copy(
                        out_v.at[:, pl.ds(0, tail)],
                        out_hbm.at[:, pl.ds(base, tail)])
            return 0

        lax.fori_loop(0, chunks, do_chunk, 0)

    return k(rob_flat, cols_flat, w_flat)


def kernel(inp, rest_of_brain, w_v1, idx_v1, w_lm, idx_lm):
    t, nbkg = rest_of_brain.shape
    cols = jnp.concatenate([idx_v1[:, 1], idx_lm[:, 1]])
    w = jnp.concatenate([w_v1, w_lm])
    n = cols.shape[0] // _SYN
    npad = pl.cdiv(n, _NW * _CHUNK) * _NW * _CHUNK
    cols_flat = jnp.pad(cols, (0, _SYN * (npad - n)))
    w_flat = jnp.pad(w, (0, _SYN * (npad - n)))
    out = sc_spmm(rest_of_brain.reshape(-1), cols_flat, w_flat, n, npad)
    return out.reshape(1, t, n)


# speedup vs baseline: 12.8056x; 12.6276x over previous
import functools

import jax
import jax.numpy as jnp
from jax import lax
from jax.experimental import pallas as pl
from jax.experimental.pallas import tpu as pltpu
from jax.experimental.pallas import tpu_sc as plsc

_SYN = 4
_NBKG_PAD = 128
_L = 16
_NW = 32


def _sc_deinterleave(cols_flat, w_flat, npad):
    slab = npad // _NW
    syn_slab = _SYN * slab
    mesh = plsc.VectorSubcoreMesh(core_axis_name="c", subcore_axis_name="s")

    @functools.partial(
        pl.kernel,
        mesh=mesh,
        out_type=(jax.ShapeDtypeStruct((_SYN, npad), jnp.int32),
                  jax.ShapeDtypeStruct((_SYN, npad), jnp.float32)),
        scratch_types=[
            pltpu.VMEM((syn_slab,), jnp.int32),
            pltpu.VMEM((syn_slab,), jnp.float32),
            pltpu.VMEM((_SYN, slab), jnp.int32),
            pltpu.VMEM((_SYN, slab), jnp.float32),
        ],
        compiler_params=pltpu.CompilerParams(needs_layout_passes=False),
    )
    def k(cols_hbm, w_hbm, ct_hbm, wt_hbm, cin_v, win_v, ct_v, wt_v):
        wid = lax.axis_index("s") * 2 + lax.axis_index("c")
        base = wid * syn_slab
        pltpu.sync_copy(cols_hbm.at[pl.ds(base, syn_slab)], cin_v)
        pltpu.sync_copy(w_hbm.at[pl.ds(base, syn_slab)], win_v)
        lanes = lax.iota(jnp.int32, _L)

        def m_body(m, _):
            b16 = m * _L
            idx0 = (b16 + lanes) * _SYN
            for s in range(_SYN):
                ct_v[s, pl.ds(b16, _L)] = plsc.load_gather(cin_v, [idx0 + s])
                wt_v[s, pl.ds(b16, _L)] = plsc.load_gather(win_v, [idx0 + s])
            return 0

        lax.fori_loop(0, slab // _L, m_body, 0)
        nbase = wid * slab
        pltpu.sync_copy(ct_v, ct_hbm.at[:, pl.ds(nbase, slab)])
        pltpu.sync_copy(wt_v, wt_hbm.at[:, pl.ds(nbase, slab)])

    return k(cols_flat, w_flat)


def _tc_body(ct_ref, wt_ref, rob_ref, out_ref):
    bn = ct_ref.shape[1]
    c_iota = jax.lax.broadcasted_iota(jnp.int32, (_NBKG_PAD, bn), 0)
    at = jnp.zeros((_NBKG_PAD, bn), dtype=jnp.float32)
    for s in range(_SYN):
        at = at + jnp.where(c_iota == ct_ref[s : s + 1, :],
                            wt_ref[s : s + 1, :], 0.0)
    out_ref[0] = jnp.dot(rob_ref[...], at.astype(jnp.bfloat16),
                         preferred_element_type=jnp.float32)


def _tc_spmm(rob_pad, ct, wt, n, block_n):
    t = rob_pad.shape[0]
    nb = ct.shape[1] // block_n
    return pl.pallas_call(
        _tc_body,
        grid=(nb,),
        in_specs=[
            pl.BlockSpec((_SYN, block_n), lambda i: (0, i)),
            pl.BlockSpec((_SYN, block_n), lambda i: (0, i)),
            pl.BlockSpec((t, _NBKG_PAD), lambda i: (0, 0)),
        ],
        out_specs=pl.BlockSpec((1, t, block_n), lambda i: (0, 0, i)),
        out_shape=jax.ShapeDtypeStruct((1, t, n), jnp.float32),
    )(ct, wt, rob_pad)


def kernel(inp, rest_of_brain, w_v1, idx_v1, w_lm, idx_lm, block_n=8192):
    t, nbkg = rest_of_brain.shape
    cols = jnp.concatenate([idx_v1[:, 1], idx_lm[:, 1]])
    w = jnp.concatenate([w_v1, w_lm])
    n = cols.shape[0] // _SYN
    npad = pl.cdiv(n, block_n) * block_n
    cols_flat = jnp.pad(cols, (0, _SYN * (npad - n)))
    w_flat = jnp.pad(w, (0, _SYN * (npad - n)))
    ct, wt = _sc_deinterleave(cols_flat, w_flat, npad)
    rob_pad = jnp.pad(rest_of_brain, ((0, 0), (0, _NBKG_PAD - nbkg)))
    return _tc_spmm(rob_pad.astype(jnp.bfloat16), ct, wt, n, block_n)
